# baseline (speedup 1.0000x reference)
import jax
import jax.numpy as jnp
from jax import lax
from jax.experimental import pallas as pl
from jax.experimental.pallas import tpu as pltpu
from jax.experimental.pallas import tpu_sc as plsc

_FRAME_RATE = 4096
_B = 4
_N = 4096
_NO = _N + 1
_C = 1024
_L = 16
_NTILES = 32
_ROWS_PER_TILE = 128
_R = 8
_RR = _R + 4


def _body(x_hbm, cls_hbm, abs_hbm, rel_hbm, out_hbm, xbuf, absbuf, relbuf):
    wid = lax.axis_index("s") * 2 + lax.axis_index("c")
    n_start = wid * _ROWS_PER_TILE
    is_last_tile = wid == _NTILES - 1
    n_end = jnp.where(is_last_tile, _NO, n_start + _ROWS_PER_TILE)
    nchunks = jnp.where(is_last_tile, _ROWS_PER_TILE // _R + 1,
                        _ROWS_PER_TILE // _R)

    def chunk(k, carry):
        c0 = jnp.minimum(n_start + k * _R, n_end - _R)
        c1 = c0 + _R

        @pl.when(c0 > 0)
        def _():
            for b in range(_B):
                pltpu.sync_copy(x_hbm.at[b, pl.ds(c0 - 1, _R)], xbuf.at[b])

        @pl.when(c0 == 0)
        def _():
            for b in range(_B):
                pltpu.sync_copy(cls_hbm.at[0], xbuf.at[b, pl.ds(0, 1)])
                pltpu.sync_copy(x_hbm.at[b, pl.ds(0, _R - 1)],
                                xbuf.at[b, pl.ds(1, _R - 1)])

        pltpu.sync_copy(abs_hbm.at[0, pl.ds(c0, _R)], absbuf)

        lo = jnp.maximum(_FRAME_RATE - c1, 0)
        pltpu.sync_copy(rel_hbm.at[pl.ds(lo, _RR)], relbuf)

        def row(i, carry2):
            n = c0 + i
            rrow = [jnp.maximum(_FRAME_RATE - 1 + b - n, 0) - lo
                    for b in range(_B)]
            for c in range(_C // _L):
                ds = pl.ds(c * _L, _L)
                a = absbuf[i, ds]
                for b in range(_B):
                    plsc.addupdate(xbuf.at[b, i, ds], a + relbuf[rrow[b], ds])
            return carry2

        lax.fori_loop(0, _R, row, 0, unroll=False)

        for b in range(_B):
            pltpu.sync_copy(xbuf.at[b], out_hbm.at[b, pl.ds(c0, _R)])
        return carry

    lax.fori_loop(0, nchunks, chunk, 0, unroll=False)


@jax.jit
def kernel(x, cls_token, abs_pos_embedding, rel_pos_embedding):
    mesh = plsc.VectorSubcoreMesh(core_axis_name="c", subcore_axis_name="s",
                                  num_cores=2, num_subcores=16)
    run = pl.kernel(
        _body,
        out_type=jax.ShapeDtypeStruct((_B, _NO, _C), jnp.float32),
        mesh=mesh,
        scratch_types=[
            pltpu.VMEM((_B, _R, _C), jnp.float32),
            pltpu.VMEM((_R, _C), jnp.float32),
            pltpu.VMEM((_RR, _C), jnp.float32),
        ],
        compiler_params=pltpu.CompilerParams(use_tc_tiling_on_sc=False),
    )
    return run(x, cls_token, abs_pos_embedding, rel_pos_embedding)


# speedup vs baseline: 1.1422x; 1.1422x over previous
import jax
import jax.numpy as jnp
from jax import lax
from jax.experimental import pallas as pl
from jax.experimental.pallas import tpu as pltpu
from jax.experimental.pallas import tpu_sc as plsc

_FRAME_RATE = 4096
_B = 4
_N = 4096
_NO = _N + 1
_C = 1024
_L = 16
_NTILES = 32
_ROWS_PER_TILE = 128
_R = 8
_RR = _R + 4


def _chunk_start(k, n_start, n_end):
    return jnp.minimum(n_start + k * _R, n_end - _R)


def _body(x_hbm, cls_hbm, abs_hbm, rel_hbm, out_hbm,
          xbuf, absbuf, relbuf, in_sem, out_sem):
    wid = lax.axis_index("s") * 2 + lax.axis_index("c")
    n_start = wid * _ROWS_PER_TILE
    is_last_tile = wid == _NTILES - 1
    n_end = jnp.where(is_last_tile, _NO, n_start + _ROWS_PER_TILE)
    nchunks = jnp.where(is_last_tile, _ROWS_PER_TILE // _R + 1,
                        _ROWS_PER_TILE // _R)

    def in_copies_normal(k, slot):
        c0 = _chunk_start(k, n_start, n_end)
        lo = jnp.maximum(_FRAME_RATE - (c0 + _R), 0)
        cps = [pltpu.make_async_copy(x_hbm.at[b, pl.ds(c0 - 1, _R)],
                                     xbuf.at[slot, b], in_sem)
               for b in range(_B)]
        cps.append(pltpu.make_async_copy(abs_hbm.at[0, pl.ds(c0, _R)],
                                         absbuf.at[slot], in_sem))
        cps.append(pltpu.make_async_copy(rel_hbm.at[pl.ds(lo, _RR)],
                                         relbuf.at[slot], in_sem))
        return cps

    def issue_in(k, slot):
        c0 = _chunk_start(k, n_start, n_end)

        @pl.when(c0 > 0)
        def _():
            for cp in in_copies_normal(k, slot):
                cp.start()

        @pl.when(c0 == 0)
        def _():
            for b in range(_B):
                pltpu.make_async_copy(cls_hbm.at[0],
                                      xbuf.at[slot, b, pl.ds(0, 1)],
                                      in_sem).start()
                pltpu.make_async_copy(x_hbm.at[b, pl.ds(0, _R - 1)],
                                      xbuf.at[slot, b, pl.ds(1, _R - 1)],
                                      in_sem).start()
            pltpu.make_async_copy(abs_hbm.at[0, pl.ds(0, _R)],
                                  absbuf.at[slot], in_sem).start()
            lo = _FRAME_RATE - _R
            pltpu.make_async_copy(rel_hbm.at[pl.ds(lo, _RR)],
                                  relbuf.at[slot], in_sem).start()

    def wait_in(k, slot):
        for cp in in_copies_normal(jnp.maximum(k, 1), slot):
            cp.wait()

    def out_copies(k, slot):
        c0 = _chunk_start(k, n_start, n_end)
        return [pltpu.make_async_copy(xbuf.at[slot, b],
                                      out_hbm.at[b, pl.ds(c0, _R)], out_sem)
                for b in range(_B)]

    def compute(k, slot):
        c0 = _chunk_start(k, n_start, n_end)
        lo = jnp.maximum(_FRAME_RATE - (c0 + _R), 0)

        def row(i, carry):
            n = c0 + i
            rrow = [jnp.maximum(_FRAME_RATE - 1 + b - n, 0) - lo
                    for b in range(_B)]
            for c in range(_C // _L):
                ds = pl.ds(c * _L, _L)
                a = absbuf[slot, i, ds]
                for b in range(_B):
                    plsc.addupdate(xbuf.at[slot, b, i, ds],
                                   a + relbuf[slot, rrow[b], ds])
            return carry

        lax.fori_loop(0, _R, row, 0, unroll=False)

    issue_in(0, 0)

    def chunk(k, carry):
        slot = k % 2
        wait_in(k, slot)

        @pl.when(k + 1 < nchunks)
        def _():
            @pl.when(k >= 1)
            def _():
                for cp in out_copies(k - 1, 1 - slot):
                    cp.wait()

            issue_in(k + 1, 1 - slot)

        compute(k, slot)
        for cp in out_copies(k, slot):
            cp.start()
        return carry

    lax.fori_loop(0, nchunks, chunk, 0, unroll=False)

    for cp in out_copies(nchunks - 2, nchunks % 2):
        cp.wait()
    for cp in out_copies(nchunks - 1, (nchunks - 1) % 2):
        cp.wait()


@jax.jit
def kernel(x, cls_token, abs_pos_embedding, rel_pos_embedding):
    mesh = plsc.VectorSubcoreMesh(core_axis_name="c", subcore_axis_name="s",
                                  num_cores=2, num_subcores=16)
    run = pl.kernel(
        _body,
        out_type=jax.ShapeDtypeStruct((_B, _NO, _C), jnp.float32),
        mesh=mesh,
        scratch_types=[
            pltpu.VMEM((2, _B, _R, _C), jnp.float32),
            pltpu.VMEM((2, _R, _C), jnp.float32),
            pltpu.VMEM((2, _RR, _C), jnp.float32),
            pltpu.SemaphoreType.DMA,
            pltpu.SemaphoreType.DMA,
        ],
        compiler_params=pltpu.CompilerParams(use_tc_tiling_on_sc=False),
    )
    return run(x, cls_token, abs_pos_embedding, rel_pos_embedding)


# speedup vs baseline: 1.3843x; 1.2120x over previous
import jax
import jax.numpy as jnp
from jax import lax
from jax.experimental import pallas as pl
from jax.experimental.pallas import tpu as pltpu
from jax.experimental.pallas import tpu_sc as plsc

_FRAME_RATE = 4096
_B = 4
_N = 4096
_NO = _N + 1
_C = 1024
_L = 16
_NTILES = 32
_ROWS_PER_TILE = 128
_R = 8
_RR = _R + 4


def _chunk_start(k, n_start, n_end):
    return jnp.minimum(n_start + k * _R, n_end - _R)


def _body(x_hbm, cls_hbm, abs_hbm, rel_hbm, out_hbm,
          xbuf, absbuf, relbuf, in_sem, out_sem):
    wid = lax.axis_index("s") * 2 + lax.axis_index("c")
    n_start = wid * _ROWS_PER_TILE
    is_last_tile = wid == _NTILES - 1
    n_end = jnp.where(is_last_tile, _NO, n_start + _ROWS_PER_TILE)
    nchunks = jnp.where(is_last_tile, _ROWS_PER_TILE // _R + 1,
                        _ROWS_PER_TILE // _R)

    def in_copies_normal(k, slot):
        c0 = _chunk_start(k, n_start, n_end)
        lo = jnp.maximum(_FRAME_RATE - (c0 + _R), 0)
        cps = [pltpu.make_async_copy(x_hbm.at[b, pl.ds(c0 - 1, _R)],
                                     xbuf.at[slot, b], in_sem)
               for b in range(_B)]
        cps.append(pltpu.make_async_copy(abs_hbm.at[0, pl.ds(c0, _R)],
                                         absbuf.at[slot], in_sem))
        cps.append(pltpu.make_async_copy(rel_hbm.at[pl.ds(lo, _RR)],
                                         relbuf.at[slot], in_sem))
        return cps

    def issue_in(k, slot):
        c0 = _chunk_start(k, n_start, n_end)

        @pl.when(c0 > 0)
        def _():
            for cp in in_copies_normal(k, slot):
                cp.start()

        @pl.when(c0 == 0)
        def _():
            for b in range(_B):
                pltpu.make_async_copy(cls_hbm.at[0],
                                      xbuf.at[slot, b, pl.ds(0, 1)],
                                      in_sem).start()
                pltpu.make_async_copy(x_hbm.at[b, pl.ds(0, _R - 1)],
                                      xbuf.at[slot, b, pl.ds(1, _R - 1)],
                                      in_sem).start()
            pltpu.make_async_copy(abs_hbm.at[0, pl.ds(0, _R)],
                                  absbuf.at[slot], in_sem).start()
            lo = _FRAME_RATE - _R
            pltpu.make_async_copy(rel_hbm.at[pl.ds(lo, _RR)],
                                  relbuf.at[slot], in_sem).start()

    def wait_in(k, slot):
        for cp in in_copies_normal(jnp.maximum(k, 1), slot):
            cp.wait()

    def out_copies(k, slot):
        c0 = _chunk_start(k, n_start, n_end)
        return [pltpu.make_async_copy(xbuf.at[slot, b],
                                      out_hbm.at[b, pl.ds(c0, _R)], out_sem)
                for b in range(_B)]

    def compute(k, slot):
        c0 = _chunk_start(k, n_start, n_end)
        lo = jnp.maximum(_FRAME_RATE - (c0 + _R), 0)
        rr = [[jnp.maximum(_FRAME_RATE - 1 + b - (c0 + i), 0) - lo
               for b in range(_B)] for i in range(_R)]

        @plsc.parallel_loop(0, _C // _L, unroll=2)
        def _(c):
            ds = pl.ds(c * _L, _L)
            for i in range(_R):
                a = absbuf[slot, i, ds]
                for b in range(_B):
                    plsc.addupdate(xbuf.at[slot, b, i, ds],
                                   a + relbuf[slot, rr[i][b], ds])

    issue_in(0, 0)

    def chunk(k, carry):
        slot = k % 2
        wait_in(k, slot)

        @pl.when(k + 1 < nchunks)
        def _():
            @pl.when(k >= 1)
            def _():
                for cp in out_copies(k - 1, 1 - slot):
                    cp.wait()

            issue_in(k + 1, 1 - slot)

        compute(k, slot)
        for cp in out_copies(k, slot):
            cp.start()
        return carry

    lax.fori_loop(0, nchunks, chunk, 0, unroll=False)

    for cp in out_copies(nchunks - 2, nchunks % 2):
        cp.wait()
    for cp in out_copies(nchunks - 1, (nchunks - 1) % 2):
        cp.wait()


@jax.jit
def kernel(x, cls_token, abs_pos_embedding, rel_pos_embedding):
    mesh = plsc.VectorSubcoreMesh(core_axis_name="c", subcore_axis_name="s",
                                  num_cores=2, num_subcores=16)
    run = pl.kernel(
        _body,
        out_type=jax.ShapeDtypeStruct((_B, _NO, _C), jnp.float32),
        mesh=mesh,
        scratch_types=[
            pltpu.VMEM((2, _B, _R, _C), jnp.float32),
            pltpu.VMEM((2, _R, _C), jnp.float32),
            pltpu.VMEM((2, _RR, _C), jnp.float32),
            pltpu.SemaphoreType.DMA,
            pltpu.SemaphoreType.DMA,
        ],
        compiler_params=pltpu.CompilerParams(use_tc_tiling_on_sc=False),
    )
    return run(x, cls_token, abs_pos_embedding, rel_pos_embedding)


# speedup vs baseline: 3.9241x; 2.8346x over previous
import jax
import jax.numpy as jnp
from jax import lax
from jax.experimental import pallas as pl
from jax.experimental.pallas import tpu as pltpu
from jax.experimental.pallas import tpu_sc as plsc

_FRAME_RATE = 4096
_B = 4
_N = 4096
_NO = _N + 1
_C = 1024
_L = 16
_CG = 4
_W = _C // _CG
_NG = 8
_GROWS = _N // _NG
_R = 16
_RELR = _R + 8
_NCHUNK = _GROWS // _R


def _body(x_hbm, cls_hbm, abs_hbm, rel_hbm, out_hbm,
          xrot, outbuf, absbuf, relbuf, clsbuf, in_sem, out_sem):
    wid = lax.axis_index("s") * 2 + lax.axis_index("c")
    g = wid // _CG
    cb = pl.multiple_of((wid % _CG) * _W, _W)
    n0 = pl.multiple_of(g * _GROWS, _R)

    def in_copies(k, xs, ar):
        c0 = pl.multiple_of(n0 + k * _R, _R)
        lo = pl.multiple_of(jnp.maximum(_FRAME_RATE - c0 - _R, 0), 8)
        cps = [pltpu.make_async_copy(
                   x_hbm.at[b, pl.ds(c0, _R), pl.ds(cb, _W)],
                   xrot.at[xs, b], in_sem) for b in range(_B)]
        cps.append(pltpu.make_async_copy(
            abs_hbm.at[0, pl.ds(c0, _R), pl.ds(cb, _W)],
            absbuf.at[ar], in_sem))
        cps.append(pltpu.make_async_copy(
            rel_hbm.at[pl.ds(lo, _RELR), pl.ds(cb, _W)],
            relbuf.at[ar], in_sem))
        return cps

    def out_copies(k, o):
        c0 = pl.multiple_of(n0 + k * _R, _R)
        return [pltpu.make_async_copy(
                    outbuf.at[o, b],
                    out_hbm.at[b, pl.ds(c0, _R), pl.ds(cb, _W)], out_sem)
                for b in range(_B)]

    def compute(k, xs, xp, ar, o):
        c0 = n0 + k * _R
        lo = jnp.maximum(_FRAME_RATE - c0 - _R, 0)
        rr = [[jnp.maximum(_FRAME_RATE - 1 + b - (c0 + i), 0) - lo
               for b in range(_B)] for i in range(_R)]

        @plsc.parallel_loop(0, _W // _L)
        def _(c):
            ds = pl.ds(c * _L, _L)
            for i in range(_R):
                a = absbuf[ar, i, ds]
                for b in range(_B):
                    xv = (xrot[xp, b, _R - 1, ds] if i == 0
                          else xrot[xs, b, i - 1, ds])
                    outbuf[o, b, i, ds] = xv + a + relbuf[ar, rr[i][b], ds]

        @pl.when((k == 0) & (g == 0))
        def _():
            @plsc.parallel_loop(0, _W // _L)
            def _(c):
                ds = pl.ds(c * _L, _L)
                a = absbuf[ar, 0, ds]
                for b in range(_B):
                    outbuf[o, b, 0, ds] = (clsbuf[b, 0, ds] + a
                                           + relbuf[ar, rr[0][b], ds])

    @pl.when(g > 0)
    def _():
        for b in range(_B):
            pltpu.sync_copy(x_hbm.at[b, pl.ds(pl.multiple_of(n0 - 8, 8), 8),
                                     pl.ds(cb, _W)],
                            xrot.at[2, b, pl.ds(_R - 8, 8)])

    @pl.when(g == 0)
    def _():
        for b in range(_B):
            pltpu.sync_copy(cls_hbm.at[0, pl.ds(0, 1), pl.ds(cb, _W)],
                            clsbuf.at[b])

    for cp in in_copies(0, 0, 0):
        cp.start()

    def chunk(k, carry):
        xs = k % 3
        xp = (k + 2) % 3
        ar = k % 2
        o = k % 2
        for cp in in_copies(k, xs, ar):
            cp.wait()

        @pl.when(k >= 2)
        def _():
            for cp in out_copies(k - 2, o):
                cp.wait()

        @pl.when(k + 1 < _NCHUNK)
        def _():
            for cp in in_copies(k + 1, (k + 1) % 3, (k + 1) % 2):
                cp.start()

        compute(k, xs, xp, ar, o)
        for cp in out_copies(k, o):
            cp.start()
        return carry

    lax.fori_loop(0, _NCHUNK, chunk, 0, unroll=False)

    for cp in out_copies(_NCHUNK - 2, (_NCHUNK - 2) % 2):
        cp.wait()
    for cp in out_copies(_NCHUNK - 1, (_NCHUNK - 1) % 2):
        cp.wait()

    @pl.when(g == _NG - 1)
    def _():
        pltpu.sync_copy(abs_hbm.at[0, pl.ds(_N, 1), pl.ds(cb, _W)],
                        absbuf.at[0, pl.ds(0, 1)])
        pltpu.sync_copy(rel_hbm.at[pl.ds(0, 8), pl.ds(cb, _W)],
                        relbuf.at[0, pl.ds(0, 8)])
        xlast = (_NCHUNK - 1) % 3

        @plsc.parallel_loop(0, _W // _L)
        def _(c):
            ds = pl.ds(c * _L, _L)
            a = absbuf[0, 0, ds]
            for b in range(_B):
                outbuf[0, b, 0, ds] = (xrot[xlast, b, _R - 1, ds] + a
                                       + relbuf[0, max(b - 1, 0), ds])

        for b in range(_B):
            pltpu.sync_copy(outbuf.at[0, b, pl.ds(0, 1)],
                            out_hbm.at[b, pl.ds(_N, 1), pl.ds(cb, _W)])


@jax.jit
def kernel(x, cls_token, abs_pos_embedding, rel_pos_embedding):
    mesh = plsc.VectorSubcoreMesh(core_axis_name="c", subcore_axis_name="s",
                                  num_cores=2, num_subcores=16)
    run = pl.kernel(
        _body,
        out_type=jax.ShapeDtypeStruct((_B, _NO, _C), jnp.float32),
        mesh=mesh,
        scratch_types=[
            pltpu.VMEM((3, _B, _R, _W), jnp.float32),
            pltpu.VMEM((2, _B, _R, _W), jnp.float32),
            pltpu.VMEM((2, _R, _W), jnp.float32),
            pltpu.VMEM((2, _RELR, _W), jnp.float32),
            pltpu.VMEM((_B, 1, _W), jnp.float32),
            pltpu.SemaphoreType.DMA,
            pltpu.SemaphoreType.DMA,
        ],
    )
    return run(x, cls_token, abs_pos_embedding, rel_pos_embedding)


# speedup vs baseline: 5.8948x; 1.5022x over previous
import jax
import jax.numpy as jnp
from jax import lax
from jax.experimental import pallas as pl
from jax.experimental.pallas import tpu as pltpu
from jax.experimental.pallas import tpu_sc as plsc

_FRAME_RATE = 4096
_B = 4
_N = 4096
_NO = _N + 1
_C = 1024
_L = 16
_CG = 4
_W = _C // _CG
_NG = 8
_GROWS = _N // _NG
_R = 16
_RELR = _R + 8
_NCHUNK = _GROWS // _R


def _body(x_hbm, cls_hbm, abs_hbm, rel_hbm, out_hbm,
          xrot, outbuf, absbuf, relbuf, clsbuf, in_sem, out_sem):
    wid = lax.axis_index("s") * 2 + lax.axis_index("c")
    g = wid // _CG
    cgi = wid % _CG
    cb = pl.multiple_of(cgi * _W, _W)
    ob = pl.multiple_of(cgi * 8, 8)
    n0 = pl.multiple_of(g * _GROWS, _R)

    def in_copies(k, xs, ar):
        c0 = pl.multiple_of(n0 + k * _R, _R)
        lo = pl.multiple_of(jnp.maximum(_FRAME_RATE - c0 - _R, 0), 8)
        cps = [pltpu.make_async_copy(
                   x_hbm.at[b, pl.ds(c0, _R), pl.ds(cb, _W)],
                   xrot.at[xs, b], in_sem) for b in range(_B)]
        cps.append(pltpu.make_async_copy(
            abs_hbm.at[0, pl.ds(c0, _R), pl.ds(cb, _W)],
            absbuf.at[ar], in_sem))
        cps.append(pltpu.make_async_copy(
            rel_hbm.at[pl.ds(lo, _RELR), pl.ds(cb, _W)],
            relbuf.at[ar], in_sem))
        return cps

    def out_copies(k, o):
        c0 = pl.multiple_of(n0 + k * _R, _R)
        return [pltpu.make_async_copy(
                    outbuf.at[o],
                    out_hbm.at[pl.ds(c0, _R), pl.ds(ob, 8)], out_sem)]

    def compute(k, xs, xp, ar, o):
        c0 = n0 + k * _R
        lo = jnp.maximum(_FRAME_RATE - c0 - _R, 0)
        rr = [[jnp.maximum(_FRAME_RATE - 1 + b - (c0 + i), 0) - lo
               for b in range(_B)] for i in range(_R)]

        @plsc.parallel_loop(0, _W // _L)
        def _(c):
            ds = pl.ds(c * _L, _L)
            mj = (c // 8) * _B
            ml = (c % 8) * _L
            for i in range(_R):
                a = absbuf[ar, i, ds]
                for b in range(_B):
                    xv = (xrot[xp, b, _R - 1, ds] if i == 0
                          else xrot[xs, b, i - 1, ds])
                    outbuf[o, i, mj + b, pl.ds(ml, _L)] = (
                        xv + a + relbuf[ar, rr[i][b], ds])

        @pl.when((k == 0) & (g == 0))
        def _():
            @plsc.parallel_loop(0, _W // _L)
            def _(c):
                ds = pl.ds(c * _L, _L)
                mj = (c // 8) * _B
                ml = (c % 8) * _L
                a = absbuf[ar, 0, ds]
                for b in range(_B):
                    outbuf[o, 0, mj + b, pl.ds(ml, _L)] = (
                        clsbuf[b, 0, ds] + a + relbuf[ar, rr[0][b], ds])

    @pl.when(g > 0)
    def _():
        for b in range(_B):
            pltpu.sync_copy(x_hbm.at[b, pl.ds(pl.multiple_of(n0 - 8, 8), 8),
                                     pl.ds(cb, _W)],
                            xrot.at[2, b, pl.ds(_R - 8, 8)])

    @pl.when(g == 0)
    def _():
        for b in range(_B):
            pltpu.sync_copy(cls_hbm.at[0, pl.ds(0, 1), pl.ds(cb, _W)],
                            clsbuf.at[b])

    for cp in in_copies(0, 0, 0):
        cp.start()

    def chunk(k, carry):
        xs = k % 3
        xp = (k + 2) % 3
        ar = k % 2
        o = k % 2
        for cp in in_copies(k, xs, ar):
            cp.wait()

        @pl.when(k >= 2)
        def _():
            for cp in out_copies(k - 2, o):
                cp.wait()

        @pl.when(k + 1 < _NCHUNK)
        def _():
            for cp in in_copies(k + 1, (k + 1) % 3, (k + 1) % 2):
                cp.start()

        compute(k, xs, xp, ar, o)
        for cp in out_copies(k, o):
            cp.start()
        return carry

    lax.fori_loop(0, _NCHUNK, chunk, 0, unroll=False)

    for cp in out_copies(_NCHUNK - 2, (_NCHUNK - 2) % 2):
        cp.wait()
    for cp in out_copies(_NCHUNK - 1, (_NCHUNK - 1) % 2):
        cp.wait()

    @pl.when(g == _NG - 1)
    def _():
        pltpu.sync_copy(abs_hbm.at[0, pl.ds(_N, 1), pl.ds(cb, _W)],
                        absbuf.at[0, pl.ds(0, 1)])
        pltpu.sync_copy(rel_hbm.at[pl.ds(0, 8), pl.ds(cb, _W)],
                        relbuf.at[0, pl.ds(0, 8)])
        xlast = (_NCHUNK - 1) % 3

        @plsc.parallel_loop(0, _W // _L)
        def _(c):
            ds = pl.ds(c * _L, _L)
            mj = (c // 8) * _B
            ml = (c % 8) * _L
            a = absbuf[0, 0, ds]
            for b in range(_B):
                outbuf[0, 0, mj + b, pl.ds(ml, _L)] = (
                    xrot[xlast, b, _R - 1, ds] + a
                    + relbuf[0, max(b - 1, 0), ds])

        pltpu.sync_copy(outbuf.at[0, pl.ds(0, 1)],
                        out_hbm.at[pl.ds(_N, 1), pl.ds(ob, 8)])


@jax.jit
def kernel(x, cls_token, abs_pos_embedding, rel_pos_embedding):
    mesh = plsc.VectorSubcoreMesh(core_axis_name="c", subcore_axis_name="s",
                                  num_cores=2, num_subcores=16)
    run = pl.kernel(
        _body,
        out_type=jax.ShapeDtypeStruct((_NO, _B * 8, 128), jnp.float32),
        mesh=mesh,
        scratch_types=[
            pltpu.VMEM((3, _B, _R, _W), jnp.float32),
            pltpu.VMEM((2, _R, 8, 128), jnp.float32),
            pltpu.VMEM((2, _R, _W), jnp.float32),
            pltpu.VMEM((2, _RELR, _W), jnp.float32),
            pltpu.VMEM((_B, 1, _W), jnp.float32),
            pltpu.SemaphoreType.DMA,
            pltpu.SemaphoreType.DMA,
        ],
    )
    out32 = run(x, cls_token, abs_pos_embedding, rel_pos_embedding)
    return (out32.reshape(_NO, 8, _B, 128)
            .transpose(2, 0, 1, 3)
            .reshape(_B, _NO, _C))
